# Initial kernel scaffold; baseline (speedup 1.0000x reference)
import jax
import jax.numpy as jnp
from jax.experimental import pallas as pl


def kernel(x, edge_index, edge_attr, W_e, b_e, W_mlp, b_mlp, gamma, beta):
    raise NotImplementedError("write your pallas kernel here")


# speedup vs baseline: 2.6660x; 2.6660x over previous
import functools

import jax
import jax.numpy as jnp
from jax import lax
from jax.experimental import pallas as pl
from jax.experimental.pallas import tpu as pltpu
from jax.experimental.pallas import tpu_sc as plsc

N_NODES = 10000
N_EDGES = 320000
D = 128
D_EDGE = 16
BN_EPS = 1e-5

NC = 2
NS = 16
NW = NC * NS
EPW = N_EDGES // NW
C = 80
NCHUNK = EPW // C
N_PAD = 10240
ROWS_PT = N_PAD // NS
ZROWS = 128
LANES = 16


_BLK1 = 3200


def _edge_proj_body(a_ref, w_ref, b_ref, o_ref):
    o_ref[...] = (
        jnp.dot(a_ref[...], w_ref[...], preferred_element_type=jnp.float32)
        + b_ref[...]
    )


def _edge_proj(edge_attr, W_e, b_e):
    return pl.pallas_call(
        _edge_proj_body,
        grid=(N_EDGES // _BLK1,),
        in_specs=[
            pl.BlockSpec((_BLK1, D_EDGE), lambda i: (i, 0)),
            pl.BlockSpec((D_EDGE, D), lambda i: (0, 0)),
            pl.BlockSpec((1, D), lambda i: (0, 0)),
        ],
        out_specs=pl.BlockSpec((_BLK1, D), lambda i: (i, 0)),
        out_shape=jax.ShapeDtypeStruct((N_EDGES, D), jnp.float32),
    )(edge_attr, W_e, b_e.reshape(1, D))


def _sc_body(x_hbm, src_hbm, dst_hbm, e_hbm, out_hbm,
             srcb, dstb, msgb, zbuf, agg_sh, gsem):
    cid = lax.axis_index("c")
    sid = lax.axis_index("s")
    wid = cid * NS + sid
    woff = wid * EPW

    zero = jnp.zeros((LANES,), jnp.float32)

    def _zrow(r, carry):
        for j in range(D // LANES):
            zbuf[r, pl.ds(j * LANES, LANES)] = zero
        return carry

    lax.fori_loop(0, ZROWS, _zrow, 0)
    for i in range(ROWS_PT // ZROWS):
        pltpu.sync_copy(zbuf, agg_sh.at[pl.ds(sid * ROWS_PT + i * ZROWS, ZROWS)])
    plsc.subcore_barrier()

    def _chunk(k, carry):
        base = pl.multiple_of(woff + k * C, 8)
        pltpu.sync_copy(src_hbm.at[pl.ds(base, C)], srcb)
        pltpu.sync_copy(dst_hbm.at[pl.ds(base, C)], dstb)
        pltpu.sync_copy(e_hbm.at[pl.ds(base, C)], msgb)
        pltpu.async_copy(x_hbm.at[srcb], msgb, gsem, add=True).wait()

        def _relu_row(r, c2):
            for j in range(D // LANES):
                sl = pl.ds(j * LANES, LANES)
                msgb[r, sl] = jnp.maximum(msgb[r, sl], 0.0)
            return c2

        lax.fori_loop(0, C, _relu_row, 0)
        pltpu.sync_copy(msgb, agg_sh.at[dstb], add=True)
        return carry

    lax.fori_loop(0, NCHUNK, _chunk, 0)

    plsc.subcore_barrier()
    pltpu.sync_copy(
        agg_sh.at[pl.ds(sid * ROWS_PT, ROWS_PT)],
        out_hbm.at[pl.ds((cid * NS + sid) * ROWS_PT, ROWS_PT)],
    )


def _segment_msgsum(x, src, dst, e):
    mesh = plsc.VectorSubcoreMesh(core_axis_name="c", subcore_axis_name="s")
    fn = pl.kernel(
        _sc_body,
        out_type=jax.ShapeDtypeStruct((NC * N_PAD, D), jnp.float32),
        mesh=mesh,
        scratch_types=[
            pltpu.VMEM((C,), jnp.int32),
            pltpu.VMEM((C,), jnp.int32),
            pltpu.VMEM((C, D), jnp.float32),
            pltpu.VMEM((ZROWS, D), jnp.float32),
            pltpu.VMEM_SHARED((N_PAD, D), jnp.float32),
            pltpu.SemaphoreType.DMA,
        ],
    )
    return fn(x, src, dst, e)


def _update_body(x_ref, agg_ref, w_ref, b_ref, g_ref, be_ref, o_ref):
    h = (x_ref[...] + agg_ref[:N_NODES, :]
         + agg_ref[N_PAD:N_PAD + N_NODES, :])
    h = jnp.dot(h, w_ref[...], preferred_element_type=jnp.float32) + b_ref[...]
    mean = jnp.mean(h, axis=0, keepdims=True)
    dlt = h - mean
    var = jnp.mean(dlt * dlt, axis=0, keepdims=True)
    h = dlt * lax.rsqrt(var + BN_EPS) * g_ref[...] + be_ref[...]
    o_ref[...] = jnp.maximum(h, 0.0)


def _node_update(x, agg, W_mlp, b_mlp, gamma, beta):
    return pl.pallas_call(
        _update_body,
        out_shape=jax.ShapeDtypeStruct((N_NODES, D), jnp.float32),
    )(x, agg, W_mlp, b_mlp.reshape(1, D), gamma.reshape(1, D),
      beta.reshape(1, D))


def kernel(x, edge_index, edge_attr, W_e, b_e, W_mlp, b_mlp, gamma, beta):
    src = edge_index[0].astype(jnp.int32)
    dst = edge_index[1].astype(jnp.int32)
    e = _edge_proj(edge_attr, W_e, b_e)
    agg = _segment_msgsum(x, src, dst, e)
    return _node_update(x, agg, W_mlp, b_mlp, gamma, beta)
